# baseline (speedup 1.0000x reference)
import functools

import jax
import jax.numpy as jnp
from jax import lax
from jax.experimental import pallas as pl
from jax.experimental.pallas import tpu as pltpu
from jax.experimental.pallas import tpu_sc as plsc

NC = 2
NS = 16
NW = NC * NS
CH = 100


def kernel(x, token_table, pos_table):
    B, T = x.shape
    D = token_table.shape[1]
    total = B * T
    rows_w = total // NW
    nchunks = rows_w // CH
    x3 = x.reshape(NW, nchunks, CH)

    mesh = plsc.VectorSubcoreMesh(
        core_axis_name="c", subcore_axis_name="s", num_cores=NC, num_subcores=NS
    )

    @functools.partial(
        pl.kernel,
        out_type=jax.ShapeDtypeStruct((NW, nchunks, CH, D), jnp.float32),
        mesh=mesh,
        compiler_params=pltpu.CompilerParams(use_tc_tiling_on_sc=False),
        scratch_types=[
            pltpu.VMEM((nchunks, CH), jnp.int32),
            pltpu.VMEM((2 * CH, D), jnp.float32),
            pltpu.VMEM((CH, D), jnp.float32),
            pltpu.SemaphoreType.DMA,
        ],
    )
    def run(x_hbm, tok_hbm, pos_hbm, out_hbm, idx_v, pos_v, buf, sem):
        wid = lax.axis_index("s") * NC + lax.axis_index("c")
        pltpu.sync_copy(x_hbm.at[wid], idx_v)
        pltpu.sync_copy(pos_hbm, pos_v)

        RU = 4

        def seq_body(i, carry):
            for p in range(2):
                j = 2 * i + p
                pltpu.async_copy(tok_hbm.at[idx_v.at[j]], buf, sem).wait()

                def add_rows(r2, c2):
                    for rr in range(RU):
                        row = RU * r2 + rr
                        for c in range(D // 16):
                            sl = pl.ds(c * 16, 16)
                            plsc.addupdate(buf.at[row, sl], pos_v[p * CH + row, sl])
                    return c2

                lax.fori_loop(0, CH // RU, add_rows, 0)
                pltpu.sync_copy(buf, out_hbm.at[wid, j])
            return carry

        lax.fori_loop(0, nchunks // 2, seq_body, 0)

    out = run(x3, token_table, pos_table)
    return out.reshape(B, T, D)


# speedup vs baseline: 1.2175x; 1.2175x over previous
import functools

import jax
import jax.numpy as jnp
from jax import lax
from jax.experimental import pallas as pl
from jax.experimental.pallas import tpu as pltpu
from jax.experimental.pallas import tpu_sc as plsc

NC = 2
NS = 16
NW = NC * NS
SEQ = 200
HALF = 100
NB = 4
PF = 2


def kernel(x, token_table, pos_table):
    B, T = x.shape
    D = token_table.shape[1]
    nchunks = (B * T) // (NW * SEQ)
    x4 = x.reshape(NW, nchunks, 2, HALF)

    mesh = plsc.VectorSubcoreMesh(
        core_axis_name="c", subcore_axis_name="s", num_cores=NC, num_subcores=NS
    )

    @functools.partial(
        pl.kernel,
        out_type=jax.ShapeDtypeStruct((NW, nchunks, SEQ, D), jnp.float32),
        mesh=mesh,
        compiler_params=pltpu.CompilerParams(use_tc_tiling_on_sc=False),
        scratch_types=[
            pltpu.VMEM((nchunks, 2, HALF), jnp.int32),
            pltpu.VMEM((SEQ, D), jnp.float32),
        ]
        + [pltpu.VMEM((SEQ, D), jnp.float32) for _ in range(NB)]
        + [pltpu.SemaphoreType.DMA for _ in range(2 * NB)],
    )
    def run(x_hbm, tok_hbm, pos_hbm, out_hbm, idx_v, pos_v, *bufs_and_sems):
        bufs = bufs_and_sems[:NB]
        gsem = bufs_and_sems[NB:2 * NB]
        ssem = bufs_and_sems[2 * NB:3 * NB]
        wid = lax.axis_index("s") * NC + lax.axis_index("c")
        pltpu.sync_copy(x_hbm.at[wid], idx_v)
        pltpu.sync_copy(pos_hbm, pos_v)

        def gstart(j, b):
            for s in range(2):
                pltpu.async_copy(
                    tok_hbm.at[idx_v.at[j, s]],
                    bufs[b].at[pl.ds(s * HALF, HALF)],
                    gsem[b],
                )

        def gwait(j, b):
            for s in range(2):
                pltpu.make_async_copy(
                    tok_hbm.at[idx_v.at[j, s]],
                    bufs[b].at[pl.ds(s * HALF, HALF)],
                    gsem[b],
                ).wait()

        def sstart(j, b):
            pltpu.async_copy(bufs[b], out_hbm.at[wid, j], ssem[b])

        def swait(b):
            pltpu.make_async_copy(bufs[b], out_hbm.at[wid, 0], ssem[b]).wait()

        RU = 8

        def add_pos(b):
            def rows(r2, c2):
                for rr in range(RU):
                    row = RU * r2 + rr
                    for c in range(D // 16):
                        sl = pl.ds(c * 16, 16)
                        plsc.addupdate(bufs[b].at[row, sl], pos_v[row, sl])
                return c2

            lax.fori_loop(0, SEQ // RU, rows, 0)

        def consume(j, b):
            gwait(j, b)
            add_pos(b)
            sstart(j, b)

        gstart(0, 0)
        gstart(1, 1)
        consume(0, 0)
        gstart(2, 2)
        consume(1, 1)
        gstart(3, 3)
        consume(2, 2)
        swait(0)
        gstart(4, 0)
        consume(3, 3)
        swait(1)
        gstart(5, 1)

        def group(g, c2):
            j0 = NB * g
            for r in range(NB):
                j = j0 + r
                b = r
                bp = (r + PF) % NB
                consume(j, b)
                swait(bp)
                gstart(j + PF, bp)
            return c2

        lax.fori_loop(1, nchunks // NB - 1, group, 0)

        j0 = nchunks - NB
        consume(j0, 0)
        swait(2)
        gstart(j0 + 2, 2)
        consume(j0 + 1, 1)
        swait(3)
        gstart(j0 + 3, 3)
        consume(j0 + 2, 2)
        consume(j0 + 3, 3)
        for b in range(NB):
            swait(b)

    out = run(x4, token_table, pos_table)
    return out.reshape(B, T, D)


# speedup vs baseline: 1.2206x; 1.0026x over previous
import functools

import jax
import jax.numpy as jnp
from jax import lax
from jax.experimental import pallas as pl
from jax.experimental.pallas import tpu as pltpu
from jax.experimental.pallas import tpu_sc as plsc

NC = 2
NS = 16
NW = NC * NS
SEQ = 200
HALF = 100
NB = 4
PF = 2


def kernel(x, token_table, pos_table):
    B, T = x.shape
    D = token_table.shape[1]
    nchunks = (B * T) // (NW * SEQ)
    x4 = x.reshape(NW, nchunks, 2, HALF)

    mesh = plsc.VectorSubcoreMesh(
        core_axis_name="c", subcore_axis_name="s", num_cores=NC, num_subcores=NS
    )

    @functools.partial(
        pl.kernel,
        out_type=jax.ShapeDtypeStruct((NW, nchunks, SEQ, D), jnp.float32),
        mesh=mesh,
        compiler_params=pltpu.CompilerParams(
            use_tc_tiling_on_sc=False, skip_device_barrier=True
        ),
        scratch_types=[
            pltpu.VMEM((nchunks, 2, HALF), jnp.int32),
            pltpu.VMEM((SEQ, D), jnp.float32),
        ]
        + [pltpu.VMEM((SEQ, D), jnp.float32) for _ in range(NB)]
        + [pltpu.SemaphoreType.DMA for _ in range(2 * NB)],
    )
    def run(x_hbm, tok_hbm, pos_hbm, out_hbm, idx_v, pos_v, *bufs_and_sems):
        bufs = bufs_and_sems[:NB]
        gsem = bufs_and_sems[NB:2 * NB]
        ssem = bufs_and_sems[2 * NB:3 * NB]
        wid = lax.axis_index("s") * NC + lax.axis_index("c")
        pltpu.sync_copy(x_hbm.at[wid], idx_v)
        pltpu.sync_copy(pos_hbm, pos_v)

        def gstart(j, b):
            for s in range(2):
                pltpu.async_copy(
                    tok_hbm.at[idx_v.at[j, s]],
                    bufs[b].at[pl.ds(s * HALF, HALF)],
                    gsem[b],
                )

        def gwait(j, b):
            for s in range(2):
                pltpu.make_async_copy(
                    tok_hbm.at[idx_v.at[j, s]],
                    bufs[b].at[pl.ds(s * HALF, HALF)],
                    gsem[b],
                ).wait()

        def sstart(j, b):
            pltpu.async_copy(bufs[b], out_hbm.at[wid, j], ssem[b])

        def swait(b):
            pltpu.make_async_copy(bufs[b], out_hbm.at[wid, 0], ssem[b]).wait()

        RU = 8

        def add_pos(b):
            def rows(r2, c2):
                for rr in range(RU):
                    row = RU * r2 + rr
                    for c in range(D // 16):
                        sl = pl.ds(c * 16, 16)
                        plsc.addupdate(bufs[b].at[row, sl], pos_v[row, sl])
                return c2

            lax.fori_loop(0, SEQ // RU, rows, 0)

        def consume(j, b):
            gwait(j, b)
            add_pos(b)
            sstart(j, b)

        gstart(0, 0)
        gstart(1, 1)
        consume(0, 0)
        gstart(2, 2)
        consume(1, 1)
        gstart(3, 3)
        consume(2, 2)
        swait(0)
        gstart(4, 0)
        consume(3, 3)
        swait(1)
        gstart(5, 1)

        def group(g, c2):
            j0 = NB * g
            for r in range(NB):
                j = j0 + r
                b = r
                bp = (r + PF) % NB
                consume(j, b)
                swait(bp)
                gstart(j + PF, bp)
            return c2

        lax.fori_loop(1, nchunks // NB - 1, group, 0)

        j0 = nchunks - NB
        consume(j0, 0)
        swait(2)
        gstart(j0 + 2, 2)
        consume(j0 + 1, 1)
        swait(3)
        gstart(j0 + 3, 3)
        consume(j0 + 2, 2)
        consume(j0 + 3, 3)
        for b in range(NB):
            swait(b)

    out = run(x4, token_table, pos_table)
    return out.reshape(B, T, D)


# speedup vs baseline: 1.2244x; 1.0031x over previous
import functools

import jax
import jax.numpy as jnp
from jax import lax
from jax.experimental import pallas as pl
from jax.experimental.pallas import tpu as pltpu
from jax.experimental.pallas import tpu_sc as plsc

NC = 2
NS = 16
NW = NC * NS
SPLITS = ((0, 96), (96, 104))
NB = 4
PF = 2


def kernel(x, token_table, pos_table):
    B, T = x.shape
    D = token_table.shape[1]
    nchunks = B // NW

    mesh = plsc.VectorSubcoreMesh(
        core_axis_name="c", subcore_axis_name="s", num_cores=NC, num_subcores=NS
    )

    @functools.partial(
        pl.kernel,
        out_type=jax.ShapeDtypeStruct((B, T, D), jnp.float32),
        mesh=mesh,
        compiler_params=pltpu.CompilerParams(use_tc_tiling_on_sc=False),
        scratch_types=[
            pltpu.VMEM((nchunks, T), jnp.int32),
            pltpu.VMEM((T, D), jnp.float32),
        ]
        + [pltpu.VMEM((T, D), jnp.float32) for _ in range(NB)]
        + [pltpu.SemaphoreType.DMA for _ in range(2 * NB)],
    )
    def run(x_hbm, tok_hbm, pos_hbm, out_hbm, idx_v, pos_v, *bufs_and_sems):
        bufs = bufs_and_sems[:NB]
        gsem = bufs_and_sems[NB:2 * NB]
        ssem = bufs_and_sems[2 * NB:3 * NB]
        wid = lax.axis_index("s") * NC + lax.axis_index("c")
        seq0 = wid * nchunks
        pltpu.sync_copy(x_hbm.at[pl.ds(seq0, nchunks)], idx_v)
        pltpu.sync_copy(pos_hbm, pos_v)

        def gstart(j, b):
            for off, n in SPLITS:
                pltpu.async_copy(
                    tok_hbm.at[idx_v.at[j, pl.ds(off, n)]],
                    bufs[b].at[pl.ds(off, n)],
                    gsem[b],
                )

        def gwait(j, b):
            for off, n in SPLITS:
                pltpu.make_async_copy(
                    tok_hbm.at[idx_v.at[j, pl.ds(off, n)]],
                    bufs[b].at[pl.ds(off, n)],
                    gsem[b],
                ).wait()

        def sstart(j, b):
            pltpu.async_copy(bufs[b], out_hbm.at[seq0 + j], ssem[b])

        def swait(b):
            pltpu.make_async_copy(bufs[b], out_hbm.at[0], ssem[b]).wait()

        RU = 8

        def add_pos(b):
            def rows(r2, c2):
                for rr in range(RU):
                    row = RU * r2 + rr
                    for c in range(D // 16):
                        sl = pl.ds(c * 16, 16)
                        plsc.addupdate(bufs[b].at[row, sl], pos_v[row, sl])
                return c2

            lax.fori_loop(0, T // RU, rows, 0)

        def consume(j, b):
            gwait(j, b)
            add_pos(b)
            sstart(j, b)

        gstart(0, 0)
        gstart(1, 1)
        consume(0, 0)
        gstart(2, 2)
        consume(1, 1)
        gstart(3, 3)
        consume(2, 2)
        swait(0)
        gstart(4, 0)
        consume(3, 3)
        swait(1)
        gstart(5, 1)

        def group(g, c2):
            j0 = NB * g
            for r in range(NB):
                j = j0 + r
                bp = (r + PF) % NB
                consume(j, r)
                swait(bp)
                gstart(j + PF, bp)
            return c2

        lax.fori_loop(1, nchunks // NB - 1, group, 0)

        j0 = nchunks - NB
        consume(j0, 0)
        swait(2)
        gstart(j0 + 2, 2)
        consume(j0 + 1, 1)
        swait(3)
        gstart(j0 + 3, 3)
        consume(j0 + 2, 2)
        consume(j0 + 3, 3)
        for b in range(NB):
            swait(b)

    return run(x, token_table, pos_table)
